# baseline (speedup 1.0000x reference)
import functools

import numpy as np

import jax
import jax.numpy as jnp
from jax import lax
from jax.experimental import pallas as pl
from jax.experimental.pallas import tpu as pltpu
from jax.experimental.pallas import tpu_sc as plsc

_Ns = 1000
_bd = 20.0
_bm = 0.1

B = 256
BB = 64

_info = plsc.get_sparse_core_info()
_NC, _NS = _info.num_cores, _info.num_subcores
_NW = _NC * _NS
_RPW = B // _NW


def _np_tables():
    ts = np.linspace(1e-05, 1.0, _Ns, dtype=np.float64)
    betas = (_bm + (_bd - _bm) * ts) / _Ns
    alphas = 1.0 - betas
    acp = np.cumprod(alphas)
    acp_prev = np.concatenate([np.ones((1,), np.float64), acp[:-1]])
    pv = betas * (1.0 - acp_prev) / (1.0 - acp)
    plv = np.log(np.clip(pv, 1e-20, None))
    c1 = betas * np.sqrt(acp_prev) / (1.0 - acp)
    c2 = (1.0 - acp_prev) * np.sqrt(alphas) / (1.0 - acp)
    return (pv.astype(np.float32), plv.astype(np.float32),
            c1.astype(np.float32), c2.astype(np.float32))


_PV, _PLV, _C1, _C2 = _np_tables()
_VTAB = np.zeros((_Ns, 128), np.float32)
_VTAB[:, 0] = _PV
_VTAB[:, 1] = _PLV


def _mean_body(i_ref, c1_ref, c2_ref, x_ref, y_ref, o_ref):
    b = pl.program_id(0)
    idx = [i_ref[b * BB + r] for r in range(BB)]
    c1v = jnp.stack([c1_ref[t] for t in idx]).reshape(BB, 1, 1, 1)
    c2v = jnp.stack([c2_ref[t] for t in idx]).reshape(BB, 1, 1, 1)
    o_ref[...] = c1v * x_ref[...] + c2v * y_ref[...]


def _sc_var_body(i_hbm, tab_hbm, out_hbm, idx_v, rows_v, sem):
    wid = lax.axis_index("s") * _NC + lax.axis_index("c")
    base = wid * _RPW
    pltpu.sync_copy(i_hbm.at[pl.ds(base, _RPW)], idx_v)
    pltpu.async_copy(tab_hbm.at[idx_v], rows_v, sem).wait()
    pltpu.sync_copy(rows_v, out_hbm.at[pl.ds(base, _RPW)])


_sc_var = functools.partial(
    pl.kernel,
    mesh=plsc.VectorSubcoreMesh(core_axis_name="c", subcore_axis_name="s"),
    out_type=jax.ShapeDtypeStruct((B, 128), jnp.float32),
    scratch_types=[
        pltpu.VMEM((_RPW,), jnp.int32),
        pltpu.VMEM((_RPW, 128), jnp.float32),
        pltpu.SemaphoreType.DMA,
    ],
)(_sc_var_body)


@jax.jit
def kernel(x0, x_i, i):
    var_rows = _sc_var(i, jnp.asarray(_VTAB))

    smem = pl.BlockSpec(memory_space=pltpu.SMEM)
    C, H, W = x0.shape[1:]
    mean = pl.pallas_call(
        _mean_body,
        grid=(B // BB,),
        in_specs=[smem, smem, smem,
                  pl.BlockSpec((BB, C, H, W), lambda b: (b, 0, 0, 0)),
                  pl.BlockSpec((BB, C, H, W), lambda b: (b, 0, 0, 0))],
        out_specs=pl.BlockSpec((BB, C, H, W), lambda b: (b, 0, 0, 0)),
        out_shape=jax.ShapeDtypeStruct(x0.shape, jnp.float32),
    )(i, jnp.asarray(_C1), jnp.asarray(_C2), x0, x_i)

    posterior_variance = var_rows[:, 0].reshape(B, 1, 1, 1)
    posterior_log_variance_clipped = var_rows[:, 1].reshape(B, 1, 1, 1)
    return (mean, posterior_variance, posterior_log_variance_clipped)


# speedup vs baseline: 3.4573x; 3.4573x over previous
import functools

import numpy as np

import jax
import jax.numpy as jnp
from jax import lax
from jax.experimental import pallas as pl
from jax.experimental.pallas import tpu as pltpu
from jax.experimental.pallas import tpu_sc as plsc

_Ns = 1000
_bd = 20.0
_bm = 0.1

B = 256
F = 3 * 64 * 64
JB = 6144

_info = plsc.get_sparse_core_info()
_NC, _NS = _info.num_cores, _info.num_subcores
_NW = _NC * _NS
_RPW = B // _NW


def _np_tables():
    ts = np.linspace(1e-05, 1.0, _Ns, dtype=np.float64)
    betas = (_bm + (_bd - _bm) * ts) / _Ns
    alphas = 1.0 - betas
    acp = np.cumprod(alphas)
    acp_prev = np.concatenate([np.ones((1,), np.float64), acp[:-1]])
    pv = betas * (1.0 - acp_prev) / (1.0 - acp)
    plv = np.log(np.clip(pv, 1e-20, None))
    c1 = betas * np.sqrt(acp_prev) / (1.0 - acp)
    c2 = (1.0 - acp_prev) * np.sqrt(alphas) / (1.0 - acp)
    return (pv.astype(np.float32), plv.astype(np.float32),
            c1.astype(np.float32), c2.astype(np.float32))


_PV, _PLV, _C1, _C2 = _np_tables()
_CTAB = np.zeros((2, 1024), np.float32)
_CTAB[0, :_Ns] = _C1
_CTAB[1, :_Ns] = _C2
_VTAB = np.zeros((_Ns, 128), np.float32)
_VTAB[:, 0] = _PV
_VTAB[:, 1] = _PLV


def _mean_body(i_ref, tab_ref, x_ref, y_ref, o_ref):
    oh = (lax.broadcasted_iota(jnp.int32, (1024, B), 0)
          == i_ref[...]).astype(jnp.float32)
    coefs = jnp.dot(tab_ref[...], oh, precision=lax.Precision.HIGHEST,
                    preferred_element_type=jnp.float32)
    o_ref[...] = coefs[0:1] * x_ref[...] + coefs[1:2] * y_ref[...]


def _sc_var_body(i_hbm, tab_hbm, out_hbm, idx_v, rows_v, sem):
    wid = lax.axis_index("s") * _NC + lax.axis_index("c")
    base = wid * _RPW
    pltpu.sync_copy(i_hbm.at[pl.ds(base, _RPW)], idx_v)
    pltpu.async_copy(tab_hbm.at[idx_v], rows_v, sem).wait()
    pltpu.sync_copy(rows_v, out_hbm.at[pl.ds(base, _RPW)])


_sc_var = functools.partial(
    pl.kernel,
    mesh=plsc.VectorSubcoreMesh(core_axis_name="c", subcore_axis_name="s"),
    out_type=jax.ShapeDtypeStruct((B, 128), jnp.float32),
    scratch_types=[
        pltpu.VMEM((_RPW,), jnp.int32),
        pltpu.VMEM((_RPW, 128), jnp.float32),
        pltpu.SemaphoreType.DMA,
    ],
)(_sc_var_body)


@jax.jit
def kernel(x0, x_i, i):
    var_rows = _sc_var(i, jnp.asarray(_VTAB))

    x0t = x0.transpose(1, 2, 3, 0).reshape(F, B)
    xit = x_i.transpose(1, 2, 3, 0).reshape(F, B)
    mean_t = pl.pallas_call(
        _mean_body,
        grid=(F // JB,),
        in_specs=[pl.BlockSpec((1, B), lambda b: (0, 0)),
                  pl.BlockSpec((2, 1024), lambda b: (0, 0)),
                  pl.BlockSpec((JB, B), lambda b: (b, 0)),
                  pl.BlockSpec((JB, B), lambda b: (b, 0))],
        out_specs=pl.BlockSpec((JB, B), lambda b: (b, 0)),
        out_shape=jax.ShapeDtypeStruct((F, B), jnp.float32),
    )(i.reshape(1, B), jnp.asarray(_CTAB), x0t, xit)

    posterior_mean = mean_t.reshape(3, 64, 64, B).transpose(3, 0, 1, 2)
    posterior_variance = var_rows[:, 0].reshape(B, 1, 1, 1)
    posterior_log_variance_clipped = var_rows[:, 1].reshape(B, 1, 1, 1)
    return (posterior_mean, posterior_variance,
            posterior_log_variance_clipped)


# speedup vs baseline: 7.1590x; 2.0707x over previous
import numpy as np

import jax
import jax.numpy as jnp
from jax import lax
from jax.experimental import pallas as pl
from jax.experimental.pallas import tpu as pltpu

_Ns = 1000
_bd = 20.0
_bm = 0.1

B = 256
F = 3 * 64 * 64
JB = 6144


def _np_tables():
    ts = np.linspace(1e-05, 1.0, _Ns, dtype=np.float64)
    betas = (_bm + (_bd - _bm) * ts) / _Ns
    alphas = 1.0 - betas
    acp = np.cumprod(alphas)
    acp_prev = np.concatenate([np.ones((1,), np.float64), acp[:-1]])
    pv = betas * (1.0 - acp_prev) / (1.0 - acp)
    plv = np.log(np.clip(pv, 1e-20, None))
    c1 = betas * np.sqrt(acp_prev) / (1.0 - acp)
    c2 = (1.0 - acp_prev) * np.sqrt(alphas) / (1.0 - acp)
    return (pv.astype(np.float32), plv.astype(np.float32),
            c1.astype(np.float32), c2.astype(np.float32))


_PV, _PLV, _C1, _C2 = _np_tables()
_TAB = np.zeros((4, 1024), np.float32)
_TAB[0, :_Ns] = _C1
_TAB[1, :_Ns] = _C2
_TAB[2, :_Ns] = _PV
_TAB[3, :_Ns] = _PLV


def _body(i_ref, tab_ref, x_ref, y_ref, o_ref, v_ref):
    b = pl.program_id(0)
    oh = (lax.broadcasted_iota(jnp.int32, (1024, B), 0)
          == i_ref[...]).astype(jnp.float32)
    coefs = jnp.dot(tab_ref[...], oh, precision=lax.Precision.HIGHEST,
                    preferred_element_type=jnp.float32)
    o_ref[...] = coefs[0:1] * x_ref[...] + coefs[1:2] * y_ref[...]

    @pl.when(b == 0)
    def _():
        v_ref[...] = jnp.concatenate(
            [coefs, jnp.zeros((4, B), jnp.float32)], axis=0)


@jax.jit
def kernel(x0, x_i, i):
    x0t = x0.transpose(1, 2, 3, 0).reshape(F, B)
    xit = x_i.transpose(1, 2, 3, 0).reshape(F, B)
    mean_t, var = pl.pallas_call(
        _body,
        grid=(F // JB,),
        in_specs=[pl.BlockSpec((1, B), lambda b: (0, 0)),
                  pl.BlockSpec((4, 1024), lambda b: (0, 0)),
                  pl.BlockSpec((JB, B), lambda b: (b, 0)),
                  pl.BlockSpec((JB, B), lambda b: (b, 0))],
        out_specs=[pl.BlockSpec((JB, B), lambda b: (b, 0)),
                   pl.BlockSpec((8, B), lambda b: (0, 0))],
        out_shape=[jax.ShapeDtypeStruct((F, B), jnp.float32),
                   jax.ShapeDtypeStruct((8, B), jnp.float32)],
    )(i.reshape(1, B), jnp.asarray(_TAB), x0t, xit)

    posterior_mean = mean_t.reshape(3, 64, 64, B).transpose(3, 0, 1, 2)
    posterior_variance = var[2].reshape(B, 1, 1, 1)
    posterior_log_variance_clipped = var[3].reshape(B, 1, 1, 1)
    return (posterior_mean, posterior_variance,
            posterior_log_variance_clipped)
